# Initial kernel scaffold; baseline (speedup 1.0000x reference)
import jax
import jax.numpy as jnp
from jax.experimental import pallas as pl


def kernel(freq, edge_index, edge_weight, conv_w, conv_b, gcn_w0, gcn_b0, gcn_w1, gcn_b1, pool_p, mlp_w0, mlp_b0, mlp_w1, mlp_b1):
    raise NotImplementedError("write your pallas kernel here")


# speedup vs baseline: 3.9676x; 3.9676x over previous
import functools

import jax
import jax.numpy as jnp
from jax import lax
from jax.experimental import pallas as pl
from jax.experimental.pallas import tpu as pltpu
from jax.experimental.pallas import tpu_sc as plsc

N = 10000
E = 320000
NP = 10240
KTOP = 5000
DUMMY = 5000
SPAD = 5120
NTILES = 32
EPT = E // NTILES
ECH = 80
NCH = EPT // ECH
ROWS_PT = NP // NTILES
ROWS_PSC = NP // 16

_mesh = plsc.VectorSubcoreMesh(core_axis_name="c", subcore_axis_name="s")
_sc_params = pltpu.CompilerParams(needs_layout_passes=False)


def _iota16():
    return lax.iota(jnp.int32, 16)


def _lane_bcast(v, r):
    s = jnp.sum(jnp.where(_iota16() == r, v, jnp.float32(0)))
    return jax.lax.broadcast_in_dim(s, (16,), ())


def _conv_stage(freq2, wmat, cb2):
    BLK = 400
    L_OUT = 113

    def body(f_ref, w_ref, b_ref, o_ref):
        f = f_ref[...]
        w = w_ref[...]
        b = b_ref[...]
        acc = jnp.zeros((BLK, 128), jnp.float32)
        for t in range(L_OUT):
            patch = jnp.concatenate(
                [f[:, c * 128 + t: c * 128 + t + 16] for c in range(4)], axis=1)
            acc = acc + jax.nn.relu(
                jnp.dot(patch, w, preferred_element_type=jnp.float32) + b)
        o_ref[...] = acc * jnp.float32(1.0 / L_OUT)

    return pl.pallas_call(
        body,
        grid=(N // BLK,),
        in_specs=[pl.BlockSpec((BLK, 512), lambda i: (i, 0)),
                  pl.BlockSpec((64, 128), lambda i: (0, 0)),
                  pl.BlockSpec((1, 128), lambda i: (0, 0))],
        out_specs=pl.BlockSpec((BLK, 128), lambda i: (i, 0)),
        out_shape=jax.ShapeDtypeStruct((N, 128), jnp.float32),
    )(freq2, wmat, cb2)


@functools.partial(
    pl.kernel,
    out_type=jax.ShapeDtypeStruct((NTILES, NP), jnp.float32),
    mesh=_mesh,
    compiler_params=_sc_params,
    scratch_types=[pltpu.VMEM((NP,), jnp.float32),
                   pltpu.VMEM((EPT,), jnp.int32),
                   pltpu.VMEM((EPT,), jnp.float32)],
)
def _deg_stage(dst_hbm, ew_hbm, out_hbm, degv, dstb, ewb):
    wid = lax.axis_index("s") * 2 + lax.axis_index("c")

    def zero_body(i, _):
        degv[pl.ds(i * 16, 16)] = jnp.zeros((16,), jnp.float32)
        return 0
    lax.fori_loop(0, NP // 16, zero_body, 0)

    e0 = wid * EPT
    pltpu.sync_copy(dst_hbm.at[pl.ds(e0, EPT)], dstb)
    pltpu.sync_copy(ew_hbm.at[pl.ds(e0, EPT)], ewb)

    def acc_body(g, _):
        idx = dstb[pl.ds(g * 16, 16)]
        w16 = ewb[pl.ds(g * 16, 16)]
        plsc.addupdate_scatter(degv, [idx], w16)
        return 0
    lax.fori_loop(0, EPT // 16, acc_body, 0)

    pltpu.sync_copy(degv, out_hbm.at[wid])


def _dis_stage(deg_p3):
    def body(d_ref, o_ref):
        deg = jnp.sum(d_ref[...], axis=0)
        dis = jnp.where(deg > 0,
                        lax.rsqrt(jnp.maximum(deg, jnp.float32(1e-12))),
                        jnp.float32(0))
        o_ref[...] = dis

    return pl.pallas_call(
        body,
        grid=(1,),
        in_specs=[pl.BlockSpec((NTILES, NP // 128, 128), lambda i: (0, 0, 0))],
        out_specs=pl.BlockSpec((NP // 128, 128), lambda i: (0, 0)),
        out_shape=jax.ShapeDtypeStruct((NP // 128, 128), jnp.float32),
    )(deg_p3)


@functools.partial(
    pl.kernel,
    out_type=jax.ShapeDtypeStruct((2, NP, 128), jnp.float32),
    mesh=_mesh,
    compiler_params=_sc_params,
    scratch_types=[pltpu.VMEM_SHARED((NP, 128), jnp.float32),
                   pltpu.VMEM((ECH, 128), jnp.float32),
                   pltpu.VMEM((NP,), jnp.float32),
                   pltpu.VMEM((ECH,), jnp.int32),
                   pltpu.VMEM((ECH,), jnp.int32),
                   pltpu.VMEM((ECH,), jnp.float32),
                   pltpu.SemaphoreType.DMA],
)
def _agg_stage(h_hbm, src_hbm, dst_hbm, ew_hbm, dis_hbm, out_hbm,
               acc, rows, disv, srcc, dstc, ewc, sem):
    cid = lax.axis_index("c")
    sid = lax.axis_index("s")
    wid = sid * 2 + cid

    def zrow(i, _):
        for j in range(8):
            rows[i, pl.ds(j * 16, 16)] = jnp.zeros((16,), jnp.float32)
        return 0
    lax.fori_loop(0, ECH, zrow, 0)
    for p in range(ROWS_PSC // ECH):
        pltpu.sync_copy(rows, acc.at[pl.ds(sid * ROWS_PSC + p * ECH, ECH)])
    plsc.subcore_barrier()

    pltpu.sync_copy(dis_hbm, disv)

    def chunk_body(k, _):
        e0 = wid * EPT + k * ECH
        pltpu.sync_copy(src_hbm.at[pl.ds(e0, ECH)], srcc)
        pltpu.sync_copy(dst_hbm.at[pl.ds(e0, ECH)], dstc)
        pltpu.sync_copy(ew_hbm.at[pl.ds(e0, ECH)], ewc)
        pltpu.async_copy(h_hbm.at[srcc], rows, sem).wait()
        for g in range(ECH // 16):
            sidx = srcc[pl.ds(g * 16, 16)]
            didx = dstc[pl.ds(g * 16, 16)]
            w16 = ewc[pl.ds(g * 16, 16)]
            c16 = (plsc.load_gather(disv, [sidx]) * w16
                   * plsc.load_gather(disv, [didx]))
            for r in range(16):
                cr = _lane_bcast(c16, r)
                row = g * 16 + r
                for j in range(8):
                    rows[row, pl.ds(j * 16, 16)] = (
                        rows[row, pl.ds(j * 16, 16)] * cr)
        pltpu.sync_copy(rows, acc.at[dstc], add=True)
        return 0
    lax.fori_loop(0, NCH, chunk_body, 0)

    plsc.subcore_barrier()
    for p in range(ROWS_PSC // ECH):
        r0 = sid * ROWS_PSC + p * ECH
        pltpu.sync_copy(acc.at[pl.ds(r0, ECH)], rows)
        pltpu.sync_copy(rows, out_hbm.at[cid, pl.ds(r0, ECH)])


def _dense_stage(parts, w, b2, relu):
    BLK = 1024

    def body(p_ref, w_ref, b_ref, o_ref):
        s = p_ref[0] + p_ref[1]
        z = jnp.dot(s, w_ref[...], preferred_element_type=jnp.float32) + b_ref[...]
        o_ref[...] = jax.nn.relu(z) if relu else z

    return pl.pallas_call(
        body,
        grid=(NP // BLK,),
        in_specs=[pl.BlockSpec((2, BLK, 128), lambda i: (0, i, 0)),
                  pl.BlockSpec((128, 128), lambda i: (0, 0)),
                  pl.BlockSpec((1, 128), lambda i: (0, 0))],
        out_specs=pl.BlockSpec((BLK, 128), lambda i: (i, 0)),
        out_shape=jax.ShapeDtypeStruct((NP, 128), jnp.float32),
    )(parts, w, b2)


def _dense2_stage(parts, w, b2, p2):
    BLK = 1024

    def body(p_ref, w_ref, b_ref, pv_ref, x_ref, s_ref):
        s = p_ref[0] + p_ref[1]
        z = jnp.dot(s, w_ref[...], preferred_element_type=jnp.float32) + b_ref[...]
        x_ref[...] = z
        s_ref[...] = lax.dot_general(
            pv_ref[...], z, (((1,), (1,)), ((), ())),
            preferred_element_type=jnp.float32)

    return pl.pallas_call(
        body,
        grid=(NP // BLK,),
        in_specs=[pl.BlockSpec((2, BLK, 128), lambda i: (0, i, 0)),
                  pl.BlockSpec((128, 128), lambda i: (0, 0)),
                  pl.BlockSpec((1, 128), lambda i: (0, 0)),
                  pl.BlockSpec((1, 128), lambda i: (0, 0))],
        out_specs=[pl.BlockSpec((BLK, 128), lambda i: (i, 0)),
                   pl.BlockSpec((1, BLK), lambda i: (0, i))],
        out_shape=[jax.ShapeDtypeStruct((NP, 128), jnp.float32),
                   jax.ShapeDtypeStruct((1, NP), jnp.float32)],
    )(parts, w, b2, p2)


def _rank_gate_stage(x2, s_t):
    BLK = 1280
    NEG = -1e30

    def body(x_ref, st_ref, y_ref, t_ref):
        pid = pl.program_id(0)
        i_idx = lax.broadcasted_iota(jnp.int32, (BLK, 1), 0) + pid * BLK
        st_blk = st_ref[:, pl.ds(pid * BLK, BLK)]
        ii = lax.broadcasted_iota(jnp.int32, (BLK, BLK), 0)
        jj = lax.broadcasted_iota(jnp.int32, (BLK, BLK), 1)
        eye = (ii == jj).astype(jnp.float32)
        s_i = lax.dot_general(eye, st_blk, (((1,), (1,)), ((), ())),
                              preferred_element_type=jnp.float32)
        s_i = jnp.where(i_idx < N, s_i, NEG)

        cnt = jnp.zeros((BLK, 128), jnp.int32)

        def jchunk(jc, cnt):
            s_j = st_ref[:, pl.ds(jc * 128, 128)]
            j_idx = lax.broadcasted_iota(jnp.int32, (1, 128), 1) + jc * 128
            s_j = jnp.where(j_idx < N, s_j, NEG)
            gt = (s_j > s_i).astype(jnp.int32)
            tie = ((s_j == s_i) & (j_idx < i_idx)).astype(jnp.int32)
            return cnt + gt + tie
        cnt = lax.fori_loop(0, NP // 128, jchunk, cnt)
        rank = jnp.sum(cnt, axis=1, keepdims=True)

        y_ref[...] = x_ref[...] * jnp.tanh(s_i)
        t_ref[...] = jnp.where(rank < KTOP, rank, DUMMY)

    return pl.pallas_call(
        body,
        grid=(NP // BLK,),
        in_specs=[pl.BlockSpec((BLK, 128), lambda i: (i, 0)),
                  pl.BlockSpec((1, NP), lambda i: (0, 0))],
        out_specs=[pl.BlockSpec((BLK, 128), lambda i: (i, 0)),
                   pl.BlockSpec((BLK, 1), lambda i: (i, 0))],
        out_shape=[jax.ShapeDtypeStruct((NP, 128), jnp.float32),
                   jax.ShapeDtypeStruct((NP, 1), jnp.int32)],
    )(x2, s_t)


@functools.partial(
    pl.kernel,
    out_type=jax.ShapeDtypeStruct((SPAD, 128), jnp.float32),
    mesh=_mesh,
    compiler_params=_sc_params,
    scratch_types=[pltpu.VMEM((ROWS_PT, 128), jnp.float32),
                   pltpu.VMEM((ECH,), jnp.int32),
                   pltpu.SemaphoreType.DMA],
)
def _scatter_stage(y_hbm, tgt_hbm, out_hbm, yv, tv, sem):
    wid = lax.axis_index("s") * 2 + lax.axis_index("c")
    pltpu.sync_copy(y_hbm.at[pl.ds(wid * ROWS_PT, ROWS_PT)], yv)
    for p in range(ROWS_PT // ECH):
        pltpu.sync_copy(tgt_hbm.at[wid, p], tv)
        pltpu.async_copy(yv.at[pl.ds(p * ECH, ECH)], out_hbm.at[tv], sem).wait()


def _mlp_stage(rowsp, w0, b02, w1, b12):
    BLK = 1024

    def body(r_ref, w0_ref, b0_ref, w1_ref, b1_ref, o_ref):
        z = jnp.dot(r_ref[...], w0_ref[...],
                    preferred_element_type=jnp.float32) + b0_ref[...]
        z = jax.nn.relu(z)
        o_ref[...] = jnp.dot(z, w1_ref[...],
                             preferred_element_type=jnp.float32) + b1_ref[...]

    return pl.pallas_call(
        body,
        grid=(SPAD // BLK,),
        in_specs=[pl.BlockSpec((BLK, 128), lambda i: (i, 0)),
                  pl.BlockSpec((128, 256), lambda i: (0, 0)),
                  pl.BlockSpec((1, 256), lambda i: (0, 0)),
                  pl.BlockSpec((256, 4), lambda i: (0, 0)),
                  pl.BlockSpec((1, 4), lambda i: (0, 0))],
        out_specs=pl.BlockSpec((BLK, 4), lambda i: (i, 0)),
        out_shape=jax.ShapeDtypeStruct((SPAD, 4), jnp.float32),
    )(rowsp, w0, b02, w1, b12)


def kernel(freq, edge_index, edge_weight, conv_w, conv_b, gcn_w0, gcn_b0,
           gcn_w1, gcn_b1, pool_p, mlp_w0, mlp_b0, mlp_w1, mlp_b1):
    freq2 = freq.reshape(N, 512)
    wmat = conv_w.transpose(1, 2, 0).reshape(64, 128)
    src = edge_index[0]
    dst = edge_index[1]

    x0 = _conv_stage(freq2, wmat, conv_b.reshape(1, 128))
    x0 = jnp.pad(x0, ((0, NP - N), (0, 0)))

    deg_p = _deg_stage(dst, edge_weight)
    dis = _dis_stage(deg_p.reshape(NTILES, NP // 128, 128)).reshape(NP)

    parts = _agg_stage(x0, src, dst, edge_weight, dis)
    h1 = _dense_stage(parts, gcn_w0, gcn_b0.reshape(1, 128), relu=True)

    parts = _agg_stage(h1, src, dst, edge_weight, dis)
    x2, s_t = _dense2_stage(parts, gcn_w1, gcn_b1.reshape(1, 128),
                            pool_p.reshape(1, 128))

    inv_pn = 1.0 / (jnp.linalg.norm(pool_p) + 1e-12)
    y, tgt = _rank_gate_stage(x2, s_t * inv_pn)

    scratch = _scatter_stage(y, tgt.reshape(NTILES, ROWS_PT // ECH, ECH))
    out = _mlp_stage(scratch, mlp_w0, mlp_b0.reshape(1, 256),
                     mlp_w1, mlp_b1.reshape(1, 4))
    return out[:KTOP]


# speedup vs baseline: 5.5005x; 1.3864x over previous
import functools

import jax
import jax.numpy as jnp
from jax import lax
from jax.experimental import pallas as pl
from jax.experimental.pallas import tpu as pltpu
from jax.experimental.pallas import tpu_sc as plsc

N = 10000
E = 320000
NP = 10240
KTOP = 5000
DUMMY = 5000
SPAD = 5120
NTILES = 32
EPT = E // NTILES
ECH = 80
NCH = EPT // ECH
ROWS_PT = NP // NTILES
ROWS_PSC = NP // 16

_mesh = plsc.VectorSubcoreMesh(core_axis_name="c", subcore_axis_name="s")
_sc_params = pltpu.CompilerParams(needs_layout_passes=False)


def _iota16():
    return lax.iota(jnp.int32, 16)


def _lane_bcast(v, r):
    s = jnp.sum(jnp.where(_iota16() == r, v, jnp.float32(0)))
    return jax.lax.broadcast_in_dim(s, (16,), ())


def _conv_stage(freq2, wmat, cb2):
    BLK = 400
    L_OUT = 113

    def body(f_ref, w_ref, b_ref, o_ref):
        f = f_ref[...]
        w = w_ref[...]
        b = b_ref[...]
        acc = jnp.zeros((BLK, 128), jnp.float32)
        for t in range(L_OUT):
            patch = jnp.concatenate(
                [f[:, c * 128 + t: c * 128 + t + 16] for c in range(4)], axis=1)
            acc = acc + jax.nn.relu(
                jnp.dot(patch, w, preferred_element_type=jnp.float32) + b)
        o_ref[...] = acc * jnp.float32(1.0 / L_OUT)

    return pl.pallas_call(
        body,
        grid=(N // BLK,),
        in_specs=[pl.BlockSpec((BLK, 512), lambda i: (i, 0)),
                  pl.BlockSpec((64, 128), lambda i: (0, 0)),
                  pl.BlockSpec((1, 128), lambda i: (0, 0))],
        out_specs=pl.BlockSpec((BLK, 128), lambda i: (i, 0)),
        out_shape=jax.ShapeDtypeStruct((N, 128), jnp.float32),
    )(freq2, wmat, cb2)


@functools.partial(
    pl.kernel,
    out_type=jax.ShapeDtypeStruct((NTILES, NP), jnp.float32),
    mesh=_mesh,
    compiler_params=_sc_params,
    scratch_types=[pltpu.VMEM((NP,), jnp.float32),
                   pltpu.VMEM((EPT,), jnp.int32),
                   pltpu.VMEM((EPT,), jnp.float32)],
)
def _deg_stage(dst_hbm, ew_hbm, out_hbm, degv, dstb, ewb):
    wid = lax.axis_index("s") * 2 + lax.axis_index("c")

    def zero_body(i, _):
        degv[pl.ds(i * 16, 16)] = jnp.zeros((16,), jnp.float32)
        return 0
    lax.fori_loop(0, NP // 16, zero_body, 0)

    e0 = wid * EPT
    pltpu.sync_copy(dst_hbm.at[pl.ds(e0, EPT)], dstb)
    pltpu.sync_copy(ew_hbm.at[pl.ds(e0, EPT)], ewb)

    def acc_body(g, _):
        idx = dstb[pl.ds(g * 16, 16)]
        w16 = ewb[pl.ds(g * 16, 16)]
        plsc.addupdate_scatter(degv, [idx], w16)
        return 0
    lax.fori_loop(0, EPT // 16, acc_body, 0)

    pltpu.sync_copy(degv, out_hbm.at[wid])


def _dis_stage(deg_p3):
    def body(d_ref, o_ref):
        deg = jnp.sum(d_ref[...], axis=0)
        dis = jnp.where(deg > 0,
                        lax.rsqrt(jnp.maximum(deg, jnp.float32(1e-12))),
                        jnp.float32(0))
        o_ref[...] = dis

    return pl.pallas_call(
        body,
        grid=(1,),
        in_specs=[pl.BlockSpec((NTILES, NP // 128, 128), lambda i: (0, 0, 0))],
        out_specs=pl.BlockSpec((NP // 128, 128), lambda i: (0, 0)),
        out_shape=jax.ShapeDtypeStruct((NP // 128, 128), jnp.float32),
    )(deg_p3)


ECA = 64
NCHA = 156
ETAIL = EPT - NCHA * ECA
GRP = 4
NGRP = NCHA // GRP
PREF = 2
G16A = ECA // 16
ACC_R = 10112
APT = ACC_R // 16
_PIECES = [(p * 64, 64) for p in range(9)] + [(576, APT - 576)]


@functools.partial(
    pl.kernel,
    out_type=jax.ShapeDtypeStruct((2, NP, 128), jnp.float32),
    mesh=_mesh,
    compiler_params=_sc_params,
    scratch_types=[pltpu.VMEM_SHARED((ACC_R, 128), jnp.float32),
                   pltpu.VMEM((GRP, ECA, 128), jnp.float32),
                   pltpu.VMEM((NP,), jnp.float32),
                   pltpu.VMEM((4 * GRP * 3, ECA), jnp.int32),
                   pltpu.VMEM((3, ETAIL), jnp.int32),
                   [pltpu.SemaphoreType.DMA] * GRP,
                   [pltpu.SemaphoreType.DMA] * GRP,
                   [pltpu.SemaphoreType.DMA] * 4],
)
def _agg_stage(h_hbm, packed_hbm, tail_hbm, dis_hbm, out_hbm,
               acc, rows, disv, iring, tailb, gsems, asems, isems):
    cid = lax.axis_index("c")
    sid = lax.axis_index("s")
    wid = sid * 2 + cid

    def zrow(i, _):
        for j in range(8):
            rows[0, i, pl.ds(j * 16, 16)] = jnp.zeros((16,), jnp.float32)
        return 0
    lax.fori_loop(0, ECA, zrow, 0)
    for off, n in _PIECES:
        pltpu.sync_copy(rows.at[0, pl.ds(0, n)],
                        acc.at[pl.ds(sid * APT + off, n)])

    pltpu.sync_copy(dis_hbm, disv)
    pltpu.sync_copy(packed_hbm.at[wid, 0], iring.at[pl.ds(0, 12)])
    pltpu.async_copy(packed_hbm.at[wid, 1], iring.at[pl.ds(12, 24 - 12)],
                     isems[1])
    pltpu.async_copy(packed_hbm.at[wid, 2], iring.at[pl.ds(24, 12)],
                     isems[2])
    plsc.subcore_barrier()

    def gather(slot, cpos, b):
        pltpu.async_copy(h_hbm.at[iring.at[slot * 12 + cpos * 3]], rows.at[b],
                         gsems[b])

    def wait_dma(sem, b):
        pltpu.make_async_copy(h_hbm.at[iring.at[0]], rows.at[b],
                              sem).wait()

    def scale(slot, cpos, b, ewsign):
        def scale_body(q, _):
            r0 = slot * 12 + cpos * 3
            sidx = iring[r0, pl.ds(q * 16, 16)]
            didx = iring[r0 + 1, pl.ds(q * 16, 16)]
            w16 = plsc.bitcast(iring[r0 + 2, pl.ds(q * 16, 16)],
                               jnp.float32)
            c16 = (plsc.load_gather(disv, [sidx]) * w16
                   * plsc.load_gather(disv, [didx]))
            for r in range(16):
                cr = jax.lax.broadcast_in_dim(c16[r], (16,), ())
                row = q * 16 + r
                for j in range(8):
                    rows[b, row, pl.ds(j * 16, 16)] = (
                        rows[b, row, pl.ds(j * 16, 16)] * cr)
            return 0
        lax.fori_loop(0, ewsign, scale_body, 0)

    for k in range(PREF):
        gather(0, k, k)

    def _wait_idx(slot):
        def mk(i):
            def f(_):
                pltpu.make_async_copy(packed_hbm.at[wid, 0],
                                      iring.at[pl.ds(i * 12, 12)],
                                      isems[i]).wait()
                return 0
            return f
        lax.switch(slot, [mk(i) for i in range(4)], 0)

    def _issue_idx(gnext, slot):
        def mk(i):
            def f(_):
                pltpu.async_copy(packed_hbm.at[wid, gnext],
                                 iring.at[pl.ds(i * 12, 12)], isems[i])
                return 0
            return f
        lax.switch(slot, [mk(i) for i in range(4)], 0)

    def group_body(g, _):
        s0 = lax.rem(g, 4)
        s1 = lax.rem(g + 1, 4)

        @pl.when(g < NGRP - 1)
        def _():
            _wait_idx(s1)

        for b in range(GRP):
            k = g * GRP + b
            wait_dma(gsems[b], b)

            if b == 2:
                @pl.when(g < NGRP - 3)
                def _(g=g):
                    _issue_idx(g + 3, lax.rem(g + 3, 4))

            @pl.when(k + PREF < NCHA)
            def _(b=b, k=k, g=g, s0=s0, s1=s1):
                bn = (b + PREF) % GRP

                @pl.when(k >= GRP - PREF)
                def _():
                    wait_dma(asems[bn], bn)
                nslot = s0 if b + PREF < GRP else s1
                gather(nslot, (b + PREF) % GRP, bn)

            scale(s0, b, b, G16A)
            pltpu.async_copy(rows.at[b],
                             acc.at[iring.at[s0 * 12 + b * 3 + 1]],
                             asems[b], add=True)
        return 0
    lax.fori_loop(0, NGRP, group_body, 0)

    for b in range(GRP):
        wait_dma(asems[b], b)

    pltpu.sync_copy(tail_hbm.at[wid, :, :], tailb)
    pltpu.async_copy(h_hbm.at[tailb.at[0]], rows.at[0, pl.ds(0, ETAIL)],
                     gsems[0]).wait()

    def tail_scale(q, _):
        sidx = tailb[0, pl.ds(q * 16, 16)]
        didx = tailb[1, pl.ds(q * 16, 16)]
        w16 = plsc.bitcast(tailb[2, pl.ds(q * 16, 16)], jnp.float32)
        c16 = (plsc.load_gather(disv, [sidx]) * w16
               * plsc.load_gather(disv, [didx]))
        for r in range(16):
            cr = jax.lax.broadcast_in_dim(c16[r], (16,), ())
            row = q * 16 + r
            for j in range(8):
                rows[0, row, pl.ds(j * 16, 16)] = (
                    rows[0, row, pl.ds(j * 16, 16)] * cr)
        return 0
    lax.fori_loop(0, ETAIL // 16, tail_scale, 0)
    pltpu.sync_copy(rows.at[0, pl.ds(0, ETAIL)], acc.at[tailb.at[1]],
                    add=True)

    plsc.subcore_barrier()
    for off, n in _PIECES:
        r0 = sid * APT + off
        pltpu.sync_copy(acc.at[pl.ds(r0, n)], rows.at[0, pl.ds(0, n)])
        pltpu.sync_copy(rows.at[0, pl.ds(0, n)], out_hbm.at[cid, pl.ds(r0, n)])


def _dense_stage(parts, w, b2, relu):
    BLK = 1024

    def body(p_ref, w_ref, b_ref, o_ref):
        s = p_ref[0] + p_ref[1]
        z = jnp.dot(s, w_ref[...], preferred_element_type=jnp.float32) + b_ref[...]
        o_ref[...] = jax.nn.relu(z) if relu else z

    return pl.pallas_call(
        body,
        grid=(NP // BLK,),
        in_specs=[pl.BlockSpec((2, BLK, 128), lambda i: (0, i, 0)),
                  pl.BlockSpec((128, 128), lambda i: (0, 0)),
                  pl.BlockSpec((1, 128), lambda i: (0, 0))],
        out_specs=pl.BlockSpec((BLK, 128), lambda i: (i, 0)),
        out_shape=jax.ShapeDtypeStruct((NP, 128), jnp.float32),
    )(parts, w, b2)


def _dense2_stage(parts, w, b2, p2):
    BLK = 1024

    def body(p_ref, w_ref, b_ref, pv_ref, x_ref, s_ref):
        s = p_ref[0] + p_ref[1]
        z = jnp.dot(s, w_ref[...], preferred_element_type=jnp.float32) + b_ref[...]
        x_ref[...] = z
        s_ref[...] = lax.dot_general(
            pv_ref[...], z, (((1,), (1,)), ((), ())),
            preferred_element_type=jnp.float32)

    return pl.pallas_call(
        body,
        grid=(NP // BLK,),
        in_specs=[pl.BlockSpec((2, BLK, 128), lambda i: (0, i, 0)),
                  pl.BlockSpec((128, 128), lambda i: (0, 0)),
                  pl.BlockSpec((1, 128), lambda i: (0, 0)),
                  pl.BlockSpec((1, 128), lambda i: (0, 0))],
        out_specs=[pl.BlockSpec((BLK, 128), lambda i: (i, 0)),
                   pl.BlockSpec((1, BLK), lambda i: (0, i))],
        out_shape=[jax.ShapeDtypeStruct((NP, 128), jnp.float32),
                   jax.ShapeDtypeStruct((1, NP), jnp.float32)],
    )(parts, w, b2, p2)


def _rank_gate_stage(x2, s_t):
    BLK = 1280
    NEG = -1e30

    def body(x_ref, st_ref, y_ref, t_ref):
        pid = pl.program_id(0)
        i_idx = lax.broadcasted_iota(jnp.int32, (BLK, 1), 0) + pid * BLK
        st_blk = st_ref[:, pl.ds(pid * BLK, BLK)]
        ii = lax.broadcasted_iota(jnp.int32, (BLK, BLK), 0)
        jj = lax.broadcasted_iota(jnp.int32, (BLK, BLK), 1)
        eye = (ii == jj).astype(jnp.float32)
        s_i = lax.dot_general(eye, st_blk, (((1,), (1,)), ((), ())),
                              preferred_element_type=jnp.float32)
        s_i = jnp.where(i_idx < N, s_i, NEG)

        cnt = jnp.zeros((BLK, 128), jnp.int32)

        def jchunk(jc, cnt):
            s_j = st_ref[:, pl.ds(jc * 128, 128)]
            j_idx = lax.broadcasted_iota(jnp.int32, (1, 128), 1) + jc * 128
            s_j = jnp.where(j_idx < N, s_j, NEG)
            gt = (s_j > s_i).astype(jnp.int32)
            tie = ((s_j == s_i) & (j_idx < i_idx)).astype(jnp.int32)
            return cnt + gt + tie
        cnt = lax.fori_loop(0, NP // 128, jchunk, cnt)
        rank = jnp.sum(cnt, axis=1, keepdims=True)

        y_ref[...] = x_ref[...] * jnp.tanh(s_i)
        t_ref[...] = jnp.where(rank < KTOP, rank, DUMMY)

    return pl.pallas_call(
        body,
        grid=(NP // BLK,),
        in_specs=[pl.BlockSpec((BLK, 128), lambda i: (i, 0)),
                  pl.BlockSpec((1, NP), lambda i: (0, 0))],
        out_specs=[pl.BlockSpec((BLK, 128), lambda i: (i, 0)),
                   pl.BlockSpec((BLK, 1), lambda i: (i, 0))],
        out_shape=[jax.ShapeDtypeStruct((NP, 128), jnp.float32),
                   jax.ShapeDtypeStruct((NP, 1), jnp.int32)],
    )(x2, s_t)


@functools.partial(
    pl.kernel,
    out_type=jax.ShapeDtypeStruct((SPAD, 128), jnp.float32),
    mesh=_mesh,
    compiler_params=_sc_params,
    scratch_types=[pltpu.VMEM((ROWS_PT, 128), jnp.float32),
                   pltpu.VMEM((ROWS_PT // ECH, ECH), jnp.int32),
                   pltpu.SemaphoreType.DMA],
)
def _scatter_stage(y_hbm, tgt_hbm, out_hbm, yv, tv, sem):
    wid = lax.axis_index("s") * 2 + lax.axis_index("c")
    pltpu.sync_copy(tgt_hbm.at[wid], tv)
    pltpu.sync_copy(y_hbm.at[pl.ds(wid * ROWS_PT, ROWS_PT)], yv)
    for p in range(ROWS_PT // ECH):
        pltpu.async_copy(yv.at[pl.ds(p * ECH, ECH)], out_hbm.at[tv.at[p]], sem)
    for p in range(ROWS_PT // ECH):
        pltpu.make_async_copy(y_hbm.at[pl.ds(wid * ROWS_PT, ECH)],
                              yv.at[pl.ds(0, ECH)], sem).wait()


def _mlp_stage(rowsp, w0, b02, w1, b12):
    BLK = 1024

    def body(r_ref, w0_ref, b0_ref, w1_ref, b1_ref, o_ref):
        z = jnp.dot(r_ref[...], w0_ref[...],
                    preferred_element_type=jnp.float32) + b0_ref[...]
        z = jax.nn.relu(z)
        o_ref[...] = jnp.dot(z, w1_ref[...],
                             preferred_element_type=jnp.float32) + b1_ref[...]

    return pl.pallas_call(
        body,
        grid=(SPAD // BLK,),
        in_specs=[pl.BlockSpec((BLK, 128), lambda i: (i, 0)),
                  pl.BlockSpec((128, 256), lambda i: (0, 0)),
                  pl.BlockSpec((1, 256), lambda i: (0, 0)),
                  pl.BlockSpec((256, 4), lambda i: (0, 0)),
                  pl.BlockSpec((1, 4), lambda i: (0, 0))],
        out_specs=pl.BlockSpec((BLK, 4), lambda i: (i, 0)),
        out_shape=jax.ShapeDtypeStruct((SPAD, 4), jnp.float32),
    )(rowsp, w0, b02, w1, b12)


def kernel(freq, edge_index, edge_weight, conv_w, conv_b, gcn_w0, gcn_b0,
           gcn_w1, gcn_b1, pool_p, mlp_w0, mlp_b0, mlp_w1, mlp_b1):
    freq2 = freq.reshape(N, 512)
    wmat = conv_w.transpose(1, 2, 0).reshape(64, 128)
    src = edge_index[0]
    dst = edge_index[1]
    ew_bits = lax.bitcast_convert_type(edge_weight, jnp.int32)
    src_t = src.reshape(NTILES, EPT)
    dst_t = dst.reshape(NTILES, EPT)
    ewb_t = ew_bits.reshape(NTILES, EPT)
    nmain = NCHA * ECA
    packed = jnp.stack(
        [src_t[:, :nmain].reshape(NTILES, NGRP, GRP, ECA),
         dst_t[:, :nmain].reshape(NTILES, NGRP, GRP, ECA),
         ewb_t[:, :nmain].reshape(NTILES, NGRP, GRP, ECA)],
        axis=3).reshape(NTILES, NGRP, GRP * 3, ECA)
    tailp = jnp.stack(
        [src_t[:, nmain:], dst_t[:, nmain:], ewb_t[:, nmain:]], axis=1)

    x0 = _conv_stage(freq2, wmat, conv_b.reshape(1, 128))
    x0 = jnp.pad(x0, ((0, NP - N), (0, 0)))

    deg_p = _deg_stage(dst, edge_weight)
    dis = _dis_stage(deg_p.reshape(NTILES, NP // 128, 128)).reshape(NP)

    parts = _agg_stage(x0, packed, tailp, dis)
    h1 = _dense_stage(parts, gcn_w0, gcn_b0.reshape(1, 128), relu=True)

    parts = _agg_stage(h1, packed, tailp, dis)
    x2, s_t = _dense2_stage(parts, gcn_w1, gcn_b1.reshape(1, 128),
                            pool_p.reshape(1, 128))

    inv_pn = 1.0 / (jnp.linalg.norm(pool_p) + 1e-12)
    y, tgt = _rank_gate_stage(x2, s_t * inv_pn)

    scratch = _scatter_stage(y, tgt.reshape(NTILES, ROWS_PT // ECH, ECH))
    out = _mlp_stage(scratch, mlp_w0, mlp_b0.reshape(1, 256),
                     mlp_w1, mlp_b1.reshape(1, 4))
    return out[:KTOP]


# speedup vs baseline: 6.8214x; 1.2401x over previous
import functools

import jax
import jax.numpy as jnp
from jax import lax
from jax.experimental import pallas as pl
from jax.experimental.pallas import tpu as pltpu
from jax.experimental.pallas import tpu_sc as plsc

N = 10000
E = 320000
NP = 10240
KTOP = 5000
DUMMY = 5000
SPAD = 5120
NTILES = 32
EPT = E // NTILES
ECH = 80
NCH = EPT // ECH
ROWS_PT = NP // NTILES
ROWS_PSC = NP // 16

_mesh = plsc.VectorSubcoreMesh(core_axis_name="c", subcore_axis_name="s")
_sc_params = pltpu.CompilerParams(needs_layout_passes=False)


def _iota16():
    return lax.iota(jnp.int32, 16)


def _lane_bcast(v, r):
    s = jnp.sum(jnp.where(_iota16() == r, v, jnp.float32(0)))
    return jax.lax.broadcast_in_dim(s, (16,), ())


def _conv_stage(freq2, w2, cb2):
    BLK = 400
    L_OUT = 113

    def body(f_ref, w_ref, b_ref, o_ref):
        f = f_ref[...]
        w = w_ref[...]
        b = b_ref[...]
        acc = jnp.zeros((BLK, 128), jnp.float32)
        for t in range(0, L_OUT - 1, 2):
            patch = jnp.concatenate(
                [f[:, c * 128 + t + d: c * 128 + t + d + 16]
                 for d in (0, 1) for c in range(4)], axis=1)
            z = jax.nn.relu(
                jnp.dot(patch, w, preferred_element_type=jnp.float32) + b)
            acc = acc + z[:, :128] + z[:, 128:]
        t = L_OUT - 1
        patch = jnp.concatenate(
            [f[:, c * 128 + t: c * 128 + t + 16] for c in range(4)], axis=1)
        acc = acc + jax.nn.relu(
            jnp.dot(patch, w[:64, :128], preferred_element_type=jnp.float32)
            + b[:, :128])
        o_ref[...] = acc * jnp.float32(1.0 / L_OUT)

    return pl.pallas_call(
        body,
        grid=(N // BLK,),
        in_specs=[pl.BlockSpec((BLK, 512), lambda i: (i, 0)),
                  pl.BlockSpec((128, 256), lambda i: (0, 0)),
                  pl.BlockSpec((1, 256), lambda i: (0, 0))],
        out_specs=pl.BlockSpec((BLK, 128), lambda i: (i, 0)),
        out_shape=jax.ShapeDtypeStruct((N, 128), jnp.float32),
    )(freq2, w2, cb2)


@functools.partial(
    pl.kernel,
    out_type=jax.ShapeDtypeStruct((NTILES, NP), jnp.float32),
    mesh=_mesh,
    compiler_params=_sc_params,
    scratch_types=[pltpu.VMEM((NP,), jnp.float32),
                   pltpu.VMEM((EPT,), jnp.int32),
                   pltpu.VMEM((EPT,), jnp.float32)],
)
def _deg_stage(dst_hbm, ew_hbm, out_hbm, degv, dstb, ewb):
    wid = lax.axis_index("s") * 2 + lax.axis_index("c")

    def zero_body(i, _):
        degv[pl.ds(i * 16, 16)] = jnp.zeros((16,), jnp.float32)
        return 0
    lax.fori_loop(0, NP // 16, zero_body, 0)

    e0 = wid * EPT
    pltpu.sync_copy(dst_hbm.at[pl.ds(e0, EPT)], dstb)
    pltpu.sync_copy(ew_hbm.at[pl.ds(e0, EPT)], ewb)

    def acc_body(g, _):
        idx = dstb[pl.ds(g * 16, 16)]
        w16 = ewb[pl.ds(g * 16, 16)]
        plsc.addupdate_scatter(degv, [idx], w16)
        return 0
    lax.fori_loop(0, EPT // 16, acc_body, 0)

    pltpu.sync_copy(degv, out_hbm.at[wid])


def _dis_stage(deg_p3):
    def body(d_ref, o_ref):
        deg = jnp.sum(d_ref[...], axis=0)
        dis = jnp.where(deg > 0,
                        lax.rsqrt(jnp.maximum(deg, jnp.float32(1e-12))),
                        jnp.float32(0))
        o_ref[...] = dis

    return pl.pallas_call(
        body,
        grid=(1,),
        in_specs=[pl.BlockSpec((NTILES, NP // 128, 128), lambda i: (0, 0, 0))],
        out_specs=pl.BlockSpec((NP // 128, 128), lambda i: (0, 0)),
        out_shape=jax.ShapeDtypeStruct((NP // 128, 128), jnp.float32),
    )(deg_p3)


ECA = 64
NCHA = 156
ETAIL = EPT - NCHA * ECA
GRP = 4
NGRP = NCHA // GRP
PREF = 2
G16A = ECA // 16
ACC_R = 10112
APT = ACC_R // 16
_PIECES = [(p * 64, 64) for p in range(9)] + [(576, APT - 576)]


@functools.partial(
    pl.kernel,
    out_type=jax.ShapeDtypeStruct((2, NP, 128), jnp.float32),
    mesh=_mesh,
    compiler_params=_sc_params,
    scratch_types=[pltpu.VMEM_SHARED((ACC_R, 128), jnp.float32),
                   pltpu.VMEM((GRP, ECA, 128), jnp.float32),
                   pltpu.VMEM((NP,), jnp.float32),
                   pltpu.VMEM((4 * GRP * 3, ECA), jnp.int32),
                   pltpu.VMEM((3, ETAIL), jnp.int32),
                   [pltpu.SemaphoreType.DMA] * GRP,
                   [pltpu.SemaphoreType.DMA] * GRP,
                   [pltpu.SemaphoreType.DMA] * 4],
)
def _agg_stage(h_hbm, packed_hbm, tail_hbm, dis_hbm, out_hbm,
               acc, rows, disv, iring, tailb, gsems, asems, isems):
    cid = lax.axis_index("c")
    sid = lax.axis_index("s")
    wid = sid * 2 + cid

    def zrow(i, _):
        for j in range(8):
            rows[0, i, pl.ds(j * 16, 16)] = jnp.zeros((16,), jnp.float32)
        return 0
    lax.fori_loop(0, ECA, zrow, 0)
    for off, n in _PIECES:
        pltpu.sync_copy(rows.at[0, pl.ds(0, n)],
                        acc.at[pl.ds(sid * APT + off, n)])

    pltpu.sync_copy(dis_hbm, disv)
    pltpu.sync_copy(packed_hbm.at[wid, 0], iring.at[pl.ds(0, 12)])
    pltpu.async_copy(packed_hbm.at[wid, 1], iring.at[pl.ds(12, 24 - 12)],
                     isems[1])
    pltpu.async_copy(packed_hbm.at[wid, 2], iring.at[pl.ds(24, 12)],
                     isems[2])
    plsc.subcore_barrier()

    def gather(slot, cpos, b):
        pltpu.async_copy(h_hbm.at[iring.at[slot * 12 + cpos * 3]], rows.at[b],
                         gsems[b])

    def wait_dma(sem, b):
        pltpu.make_async_copy(h_hbm.at[iring.at[0]], rows.at[b],
                              sem).wait()

    def scale(slot, cpos, b, ewsign):
        def scale_body(q, _):
            r0 = slot * 12 + cpos * 3
            sidx = iring[r0, pl.ds(q * 16, 16)]
            didx = iring[r0 + 1, pl.ds(q * 16, 16)]
            w16 = plsc.bitcast(iring[r0 + 2, pl.ds(q * 16, 16)],
                               jnp.float32)
            c16 = (plsc.load_gather(disv, [sidx]) * w16
                   * plsc.load_gather(disv, [didx]))
            for r in range(16):
                cr = jax.lax.broadcast_in_dim(c16[r], (16,), ())
                row = q * 16 + r
                for j in range(8):
                    rows[b, row, pl.ds(j * 16, 16)] = (
                        rows[b, row, pl.ds(j * 16, 16)] * cr)
            return 0
        lax.fori_loop(0, ewsign, scale_body, 0)

    for k in range(PREF):
        gather(0, k, k)

    def _wait_idx(slot):
        def mk(i):
            def f(_):
                pltpu.make_async_copy(packed_hbm.at[wid, 0],
                                      iring.at[pl.ds(i * 12, 12)],
                                      isems[i]).wait()
                return 0
            return f
        lax.switch(slot, [mk(i) for i in range(4)], 0)

    def _issue_idx(gnext, slot):
        def mk(i):
            def f(_):
                pltpu.async_copy(packed_hbm.at[wid, gnext],
                                 iring.at[pl.ds(i * 12, 12)], isems[i])
                return 0
            return f
        lax.switch(slot, [mk(i) for i in range(4)], 0)

    def group_body(g, _):
        s0 = lax.rem(g, 4)
        s1 = lax.rem(g + 1, 4)

        @pl.when(g < NGRP - 1)
        def _():
            _wait_idx(s1)

        for b in range(GRP):
            k = g * GRP + b
            wait_dma(gsems[b], b)

            if b == 2:
                @pl.when(g < NGRP - 3)
                def _(g=g):
                    _issue_idx(g + 3, lax.rem(g + 3, 4))

            @pl.when(k + PREF < NCHA)
            def _(b=b, k=k, g=g, s0=s0, s1=s1):
                bn = (b + PREF) % GRP

                @pl.when(k >= GRP - PREF)
                def _():
                    wait_dma(asems[bn], bn)
                nslot = s0 if b + PREF < GRP else s1
                gather(nslot, (b + PREF) % GRP, bn)

            scale(s0, b, b, G16A)
            pltpu.async_copy(rows.at[b],
                             acc.at[iring.at[s0 * 12 + b * 3 + 1]],
                             asems[b], add=True)
        return 0
    lax.fori_loop(0, NGRP, group_body, 0)

    for b in range(GRP):
        wait_dma(asems[b], b)

    pltpu.sync_copy(tail_hbm.at[wid, :, :], tailb)
    pltpu.async_copy(h_hbm.at[tailb.at[0]], rows.at[0, pl.ds(0, ETAIL)],
                     gsems[0]).wait()

    def tail_scale(q, _):
        sidx = tailb[0, pl.ds(q * 16, 16)]
        didx = tailb[1, pl.ds(q * 16, 16)]
        w16 = plsc.bitcast(tailb[2, pl.ds(q * 16, 16)], jnp.float32)
        c16 = (plsc.load_gather(disv, [sidx]) * w16
               * plsc.load_gather(disv, [didx]))
        for r in range(16):
            cr = jax.lax.broadcast_in_dim(c16[r], (16,), ())
            row = q * 16 + r
            for j in range(8):
                rows[0, row, pl.ds(j * 16, 16)] = (
                    rows[0, row, pl.ds(j * 16, 16)] * cr)
        return 0
    lax.fori_loop(0, ETAIL // 16, tail_scale, 0)
    pltpu.sync_copy(rows.at[0, pl.ds(0, ETAIL)], acc.at[tailb.at[1]],
                    add=True)

    plsc.subcore_barrier()
    for off, n in _PIECES:
        r0 = sid * APT + off
        pltpu.sync_copy(acc.at[pl.ds(r0, n)], rows.at[0, pl.ds(0, n)])
        pltpu.sync_copy(rows.at[0, pl.ds(0, n)], out_hbm.at[cid, pl.ds(r0, n)])


def _dense_stage(parts, w, b2, relu):
    BLK = 1024

    def body(p_ref, w_ref, b_ref, o_ref):
        s = p_ref[0] + p_ref[1]
        z = jnp.dot(s, w_ref[...], preferred_element_type=jnp.float32) + b_ref[...]
        o_ref[...] = jax.nn.relu(z) if relu else z

    return pl.pallas_call(
        body,
        grid=(NP // BLK,),
        in_specs=[pl.BlockSpec((2, BLK, 128), lambda i: (0, i, 0)),
                  pl.BlockSpec((128, 128), lambda i: (0, 0)),
                  pl.BlockSpec((1, 128), lambda i: (0, 0))],
        out_specs=pl.BlockSpec((BLK, 128), lambda i: (i, 0)),
        out_shape=jax.ShapeDtypeStruct((NP, 128), jnp.float32),
    )(parts, w, b2)


def _dense2_stage(parts, w, b2, p2):
    BLK = 1024

    def body(p_ref, w_ref, b_ref, pv_ref, x_ref, s_ref):
        s = p_ref[0] + p_ref[1]
        z = jnp.dot(s, w_ref[...], preferred_element_type=jnp.float32) + b_ref[...]
        x_ref[...] = z
        s_ref[...] = lax.dot_general(
            pv_ref[...], z, (((1,), (1,)), ((), ())),
            preferred_element_type=jnp.float32)

    return pl.pallas_call(
        body,
        grid=(NP // BLK,),
        in_specs=[pl.BlockSpec((2, BLK, 128), lambda i: (0, i, 0)),
                  pl.BlockSpec((128, 128), lambda i: (0, 0)),
                  pl.BlockSpec((1, 128), lambda i: (0, 0)),
                  pl.BlockSpec((1, 128), lambda i: (0, 0))],
        out_specs=[pl.BlockSpec((BLK, 128), lambda i: (i, 0)),
                   pl.BlockSpec((1, BLK), lambda i: (0, i))],
        out_shape=[jax.ShapeDtypeStruct((NP, 128), jnp.float32),
                   jax.ShapeDtypeStruct((1, NP), jnp.float32)],
    )(parts, w, b2, p2)


def _rank_gate_stage(x2, s_t):
    BLK = 1280
    NEG = -1e30

    def body(x_ref, st_ref, y_ref, t_ref):
        pid = pl.program_id(0)
        i_idx = lax.broadcasted_iota(jnp.int32, (BLK, 1), 0) + pid * BLK
        st_blk = st_ref[:, pl.ds(pid * BLK, BLK)]
        ii = lax.broadcasted_iota(jnp.int32, (BLK, BLK), 0)
        jj = lax.broadcasted_iota(jnp.int32, (BLK, BLK), 1)
        eye = (ii == jj).astype(jnp.float32)
        s_i = lax.dot_general(eye, st_blk, (((1,), (1,)), ((), ())),
                              preferred_element_type=jnp.float32)
        s_i = jnp.where(i_idx < N, s_i, NEG)

        cnt = jnp.zeros((BLK, 128), jnp.int32)

        def jchunk(jc, cnt):
            s_j = st_ref[:, pl.ds(jc * 128, 128)]
            j_idx = lax.broadcasted_iota(jnp.int32, (1, 128), 1) + jc * 128
            s_j = jnp.where(j_idx < N, s_j, NEG)
            gt = (s_j > s_i).astype(jnp.int32)
            tie = ((s_j == s_i) & (j_idx < i_idx)).astype(jnp.int32)
            return cnt + gt + tie
        cnt = lax.fori_loop(0, NP // 128, jchunk, cnt)
        rank = jnp.sum(cnt, axis=1, keepdims=True)

        y_ref[...] = x_ref[...] * jnp.tanh(s_i)
        t_ref[...] = jnp.where(rank < KTOP, rank, DUMMY)

    return pl.pallas_call(
        body,
        grid=(NP // BLK,),
        in_specs=[pl.BlockSpec((BLK, 128), lambda i: (i, 0)),
                  pl.BlockSpec((1, NP), lambda i: (0, 0))],
        out_specs=[pl.BlockSpec((BLK, 128), lambda i: (i, 0)),
                   pl.BlockSpec((BLK, 1), lambda i: (i, 0))],
        out_shape=[jax.ShapeDtypeStruct((NP, 128), jnp.float32),
                   jax.ShapeDtypeStruct((NP, 1), jnp.int32)],
    )(x2, s_t)


@functools.partial(
    pl.kernel,
    out_type=jax.ShapeDtypeStruct((SPAD, 128), jnp.float32),
    mesh=_mesh,
    compiler_params=_sc_params,
    scratch_types=[pltpu.VMEM((ROWS_PT, 128), jnp.float32),
                   pltpu.VMEM((ROWS_PT // ECH, ECH), jnp.int32),
                   pltpu.SemaphoreType.DMA],
)
def _scatter_stage(y_hbm, tgt_hbm, out_hbm, yv, tv, sem):
    wid = lax.axis_index("s") * 2 + lax.axis_index("c")
    pltpu.sync_copy(tgt_hbm.at[wid], tv)
    pltpu.sync_copy(y_hbm.at[pl.ds(wid * ROWS_PT, ROWS_PT)], yv)
    for p in range(ROWS_PT // ECH):
        pltpu.async_copy(yv.at[pl.ds(p * ECH, ECH)], out_hbm.at[tv.at[p]], sem)
    for p in range(ROWS_PT // ECH):
        pltpu.make_async_copy(y_hbm.at[pl.ds(wid * ROWS_PT, ECH)],
                              yv.at[pl.ds(0, ECH)], sem).wait()


def _mlp_stage(rowsp, w0, b02, w1, b12):
    BLK = 1024

    def body(r_ref, w0_ref, b0_ref, w1_ref, b1_ref, o_ref):
        z = jnp.dot(r_ref[...], w0_ref[...],
                    preferred_element_type=jnp.float32) + b0_ref[...]
        z = jax.nn.relu(z)
        o_ref[...] = jnp.dot(z, w1_ref[...],
                             preferred_element_type=jnp.float32) + b1_ref[...]

    return pl.pallas_call(
        body,
        grid=(SPAD // BLK,),
        in_specs=[pl.BlockSpec((BLK, 128), lambda i: (i, 0)),
                  pl.BlockSpec((128, 256), lambda i: (0, 0)),
                  pl.BlockSpec((1, 256), lambda i: (0, 0)),
                  pl.BlockSpec((256, 4), lambda i: (0, 0)),
                  pl.BlockSpec((1, 4), lambda i: (0, 0))],
        out_specs=pl.BlockSpec((BLK, 4), lambda i: (i, 0)),
        out_shape=jax.ShapeDtypeStruct((SPAD, 4), jnp.float32),
    )(rowsp, w0, b02, w1, b12)


def kernel(freq, edge_index, edge_weight, conv_w, conv_b, gcn_w0, gcn_b0,
           gcn_w1, gcn_b1, pool_p, mlp_w0, mlp_b0, mlp_w1, mlp_b1):
    freq2 = freq.reshape(N, 512)
    wmat = conv_w.transpose(1, 2, 0).reshape(64, 128)
    w2 = jnp.zeros((128, 256), jnp.float32)
    w2 = w2.at[:64, :128].set(wmat).at[64:, 128:].set(wmat)
    cb2 = jnp.concatenate([conv_b, conv_b]).reshape(1, 256)
    src = edge_index[0]
    dst = edge_index[1]
    ew_bits = lax.bitcast_convert_type(edge_weight, jnp.int32)
    src_t = src.reshape(NTILES, EPT)
    dst_t = dst.reshape(NTILES, EPT)
    ewb_t = ew_bits.reshape(NTILES, EPT)
    nmain = NCHA * ECA
    packed = jnp.stack(
        [src_t[:, :nmain].reshape(NTILES, NGRP, GRP, ECA),
         dst_t[:, :nmain].reshape(NTILES, NGRP, GRP, ECA),
         ewb_t[:, :nmain].reshape(NTILES, NGRP, GRP, ECA)],
        axis=3).reshape(NTILES, NGRP, GRP * 3, ECA)
    tailp = jnp.stack(
        [src_t[:, nmain:], dst_t[:, nmain:], ewb_t[:, nmain:]], axis=1)

    x0 = _conv_stage(freq2, w2, cb2)
    x0 = jnp.pad(x0, ((0, NP - N), (0, 0)))

    deg_p = _deg_stage(dst, edge_weight)
    dis = _dis_stage(deg_p.reshape(NTILES, NP // 128, 128)).reshape(NP)

    parts = _agg_stage(x0, packed, tailp, dis)
    h1 = _dense_stage(parts, gcn_w0, gcn_b0.reshape(1, 128), relu=True)

    parts = _agg_stage(h1, packed, tailp, dis)
    x2, s_t = _dense2_stage(parts, gcn_w1, gcn_b1.reshape(1, 128),
                            pool_p.reshape(1, 128))

    inv_pn = 1.0 / (jnp.linalg.norm(pool_p) + 1e-12)
    y, tgt = _rank_gate_stage(x2, s_t * inv_pn)

    scratch = _scatter_stage(y, tgt.reshape(NTILES, ROWS_PT // ECH, ECH))
    out = _mlp_stage(scratch, mlp_w0, mlp_b0.reshape(1, 256),
                     mlp_w1, mlp_b1.reshape(1, 4))
    return out[:KTOP]
